# baseline (speedup 1.0000x reference)
import functools

import jax
import jax.numpy as jnp
from jax import lax
from jax.experimental import pallas as pl
from jax.experimental.pallas import tpu as pltpu
from jax.experimental.pallas import tpu_sc as plsc

NUM = 16
H = 1024
W = 1024
HSC = 768
NBINS = NUM * NUM

_info = plsc.get_sparse_core_info()
NC, NS, L = _info.num_cores, _info.num_subcores, _info.num_lanes
NW = NC * NS
RPW = HSC // NW
VPR = W // 16
VPR_LOG2 = 6
NCH = 3
CR = RPW // NCH
UNROLL = 8

TBR = 128
TGRID = (H - HSC) // TBR
TSUB = TBR // NUM


def _sc_hist_kernel(pred_hbm, true_hbm, hist_out,
                    pbuf0, tbuf0, pbuf1, tbuf1, hacc, rowbuf,
                    sp0, st0, sp1, st1):
    wid = lax.axis_index("s") * NC + lax.axis_index("c")
    pbufs, tbufs = (pbuf0, pbuf1), (tbuf0, tbuf1)
    sems = ((sp0, st0), (sp1, st1))

    def start(k, slot):
        row = wid * RPW + k * CR
        hp = pltpu.async_copy(pred_hbm.at[pl.ds(row, CR)], pbufs[slot],
                              sems[slot][0])
        ht = pltpu.async_copy(true_hbm.at[pl.ds(row, CR)], tbufs[slot],
                              sems[slot][1])
        return hp, ht

    inflight = [None, None]
    inflight[0] = start(0, 0)

    zero16 = jnp.zeros((L,), jnp.float32)

    def zbody(i, _):
        hacc[pl.ds(pl.multiple_of(i * L, L), L)] = zero16
        return 0
    lax.fori_loop(0, (L * NBINS) // L, zbody, 0)

    lane_offs = lax.iota(jnp.int32, L) * NBINS
    ones = jnp.ones((L,), jnp.float32)

    for k in range(NCH):
        slot = k % 2
        if k + 1 < NCH:
            inflight[1 - slot] = start(k + 1, 1 - slot)
        hp, ht = inflight[slot]
        hp.wait()
        ht.wait()
        pb, tb = pbufs[slot], tbufs[slot]

        @plsc.parallel_loop(0, (CR * W) // L, 1, unroll=UNROLL)
        def body(i, pb=pb, tb=tb):
            r = i >> VPR_LOG2
            c = pl.multiple_of((i & (VPR - 1)) * L, L)
            p = pb[r, pl.ds(c, L)]
            t = tb[r, pl.ds(c, L)]
            comb = (p * 16.0 + t).astype(jnp.int32) + lane_offs
            plsc.addupdate_scatter(hacc, [comb], ones)

    for c in range(NBINS // L):
        def mbody(l, acc, c=c):
            return acc + hacc[pl.ds(pl.multiple_of(l * NBINS + c * L, L), L)]
        rowbuf[pl.ds(c * L, L)] = lax.fori_loop(0, L, mbody, zero16)

    pltpu.sync_copy(rowbuf, hist_out.at[wid])


_sc_hist = functools.partial(
    pl.kernel,
    mesh=plsc.VectorSubcoreMesh(core_axis_name="c", subcore_axis_name="s"),
    out_type=jax.ShapeDtypeStruct((NW, NBINS), jnp.float32),
    scratch_types=[pltpu.VMEM((CR, W), jnp.float32),
                   pltpu.VMEM((CR, W), jnp.float32),
                   pltpu.VMEM((CR, W), jnp.float32),
                   pltpu.VMEM((CR, W), jnp.float32),
                   pltpu.VMEM((L * NBINS,), jnp.float32),
                   pltpu.VMEM((NBINS,), jnp.float32),
                   pltpu.SemaphoreType.DMA,
                   pltpu.SemaphoreType.DMA,
                   pltpu.SemaphoreType.DMA,
                   pltpu.SemaphoreType.DMA],
    compiler_params=pltpu.CompilerParams(needs_layout_passes=False),
)(_sc_hist_kernel)


def _tc_hist_kernel(pred_ref, true_ref, out_ref, acc_ref):
    step = pl.program_id(0)

    @pl.when(step == 0)
    def _init():
        acc_ref[...] = jnp.zeros_like(acc_ref)

    p = pred_ref[...]
    t = true_ref[...]

    ids = (jax.lax.broadcasted_iota(jnp.int32, (NUM * NUM, W), 0)
           & (NUM - 1)).astype(jnp.float32)
    r = jnp.zeros((NUM * NUM, NUM * NUM), jnp.float32)
    for u in range(TSUB):
        ps = p[u * NUM:(u + 1) * NUM, :]
        ts = t[u * NUM:(u + 1) * NUM, :]
        pr = jnp.broadcast_to(ps[:, None, :],
                              (NUM, NUM, W)).reshape(NUM * NUM, W)
        tr = jnp.broadcast_to(ts[:, None, :],
                              (NUM, NUM, W)).reshape(NUM * NUM, W)
        ap = (pr == ids).astype(jnp.bfloat16)
        at = (tr == ids).astype(jnp.bfloat16)
        r = r + jax.lax.dot_general(ap, at, (((1,), (1,)), ((), ())),
                                    preferred_element_type=jnp.float32)
    acc_ref[...] += r

    @pl.when(step == TGRID - 1)
    def _fin():
        rm = acc_ref[...]
        s0 = jax.lax.broadcasted_iota(jnp.int32, (NBINS, NBINS), 0)
        s1 = jax.lax.broadcasted_iota(jnp.int32, (NBINS, NBINS), 1)
        rm = jnp.where((s0 >> 4) == (s1 >> 4), rm, 0.0)
        e0 = jax.lax.broadcasted_iota(jnp.int32, (NBINS, NUM), 0)
        e1 = jax.lax.broadcasted_iota(jnp.int32, (NBINS, NUM), 1)
        e = ((e0 & (NUM - 1)) == e1).astype(jnp.float32)
        re = jax.lax.dot_general(rm, e, (((1,), (0,)), ((), ())),
                                 preferred_element_type=jnp.float32)
        out_ref[...] = jax.lax.dot_general(e, re, (((0,), (0,)), ((), ())),
                                           preferred_element_type=jnp.float32)


def _tc_epilogue_kernel(hist_ref, jtc_ref, out_ref):
    h = hist_ref[...]
    flat = jnp.sum(h, axis=0, keepdims=True)

    bi = jax.lax.broadcasted_iota(jnp.int32, (NUM, NBINS), 0)
    bm = jax.lax.broadcasted_iota(jnp.int32, (NUM, NBINS), 1)
    d = jnp.where((bm >> 4) == bi, flat, 0.0)
    cm = jax.lax.broadcasted_iota(jnp.int32, (NBINS, NUM), 0)
    cj = jax.lax.broadcasted_iota(jnp.int32, (NBINS, NUM), 1)
    c = ((cm & (NUM - 1)) == cj).astype(jnp.float32)
    j = jax.lax.dot_general(d, c, (((1,), (0,)), ((), ())),
                            preferred_element_type=jnp.float32)
    j = j + jtc_ref[...]

    ri = jax.lax.broadcasted_iota(jnp.int32, (NUM, NUM), 0)
    ci = jax.lax.broadcasted_iota(jnp.int32, (NUM, NUM), 1)
    df = (ri - ci).astype(jnp.float32)
    mse_sum = jnp.sum(j * df * df)
    valid = (ri >= 1) & (ci >= 1)
    inter = jnp.where(valid, j, 0.0)
    pc = jnp.sum(j, axis=1, keepdims=True)
    tc = jnp.sum(j, axis=0, keepdims=True)
    union = pc + tc - inter
    iou = jnp.where(valid & (union != 0.0),
                    inter / jnp.maximum(union, 1e-12), 0.0)
    max_p = jnp.max(iou, axis=1, keepdims=True)
    max_t = jnp.max(iou, axis=0, keepdims=True)
    rv = (jax.lax.broadcasted_iota(jnp.int32, (NUM, 1), 0) >= 1) & (pc > 0)
    cv = (jax.lax.broadcasted_iota(jnp.int32, (1, NUM), 1) >= 1) & (tc > 0)
    loss_p = jnp.sum(jnp.where(rv, 1.0 - max_p, 0.0))
    loss_t = jnp.sum(jnp.where(cv, 1.0 - max_t, 0.0))
    ninst = (jnp.sum(rv.astype(jnp.float32))
             + jnp.sum(cv.astype(jnp.float32)))
    total = mse_sum / (H * W) / 1000.0 + loss_p + loss_t
    out_ref[...] = jnp.reshape(jnp.where(ninst == 0.0, 0.0, total), (1, 1))


def kernel(pred_mask, true_mask):
    hist = _sc_hist(pred_mask, true_mask)
    off = HSC // TBR
    jtc = pl.pallas_call(
        _tc_hist_kernel,
        grid=(TGRID,),
        in_specs=[pl.BlockSpec((TBR, W), lambda i: (i + off, 0)),
                  pl.BlockSpec((TBR, W), lambda i: (i + off, 0))],
        out_specs=pl.BlockSpec((NUM, NUM), lambda i: (0, 0)),
        out_shape=jax.ShapeDtypeStruct((NUM, NUM), jnp.float32),
        scratch_shapes=[pltpu.VMEM((NBINS, NBINS), jnp.float32)],
    )(pred_mask, true_mask)
    out = pl.pallas_call(
        _tc_epilogue_kernel,
        out_shape=jax.ShapeDtypeStruct((1, 1), jnp.float32),
    )(hist, jtc)
    return out[0, 0]


# speedup vs baseline: 1.0319x; 1.0319x over previous
import functools

import jax
import jax.numpy as jnp
from jax import lax
from jax.experimental import pallas as pl
from jax.experimental.pallas import tpu as pltpu
from jax.experimental.pallas import tpu_sc as plsc

NUM = 16
H = 1024
W = 1024
HSC = 512
NBINS = NUM * NUM

_info = plsc.get_sparse_core_info()
NC, NS, L = _info.num_cores, _info.num_subcores, _info.num_lanes
NW = NC * NS
RPW = HSC // NW
VPR = W // 16
VPR_LOG2 = 6
NCH = 1
CR = RPW // NCH
UNROLL = 4

TBR = 128
TGRID = (H - HSC) // TBR
TSUB = TBR // NUM


def _sc_hist_kernel(pred_hbm, true_hbm, hist_out,
                    pbuf0, tbuf0, pbuf1, tbuf1, hacc, rowbuf,
                    sp0, st0, sp1, st1):
    wid = lax.axis_index("s") * NC + lax.axis_index("c")
    pbufs, tbufs = (pbuf0, pbuf1), (tbuf0, tbuf1)
    sems = ((sp0, st0), (sp1, st1))

    def start(k, slot):
        row = wid * RPW + k * CR
        hp = pltpu.async_copy(pred_hbm.at[pl.ds(row, CR)], pbufs[slot],
                              sems[slot][0])
        ht = pltpu.async_copy(true_hbm.at[pl.ds(row, CR)], tbufs[slot],
                              sems[slot][1])
        return hp, ht

    inflight = [None, None]
    inflight[0] = start(0, 0)

    zero16 = jnp.zeros((L,), jnp.float32)

    def zbody(i, _):
        hacc[pl.ds(pl.multiple_of(i * L, L), L)] = zero16
        return 0
    lax.fori_loop(0, (L * NBINS) // L, zbody, 0)

    lane_offs = lax.iota(jnp.int32, L) * NBINS
    ones = jnp.ones((L,), jnp.float32)

    for k in range(NCH):
        slot = k % 2
        if k + 1 < NCH:
            inflight[1 - slot] = start(k + 1, 1 - slot)
        hp, ht = inflight[slot]
        hp.wait()
        ht.wait()
        pb, tb = pbufs[slot], tbufs[slot]

        @plsc.parallel_loop(0, (CR * W) // L, 1, unroll=UNROLL)
        def body(i, pb=pb, tb=tb):
            r = i >> VPR_LOG2
            c = pl.multiple_of((i & (VPR - 1)) * L, L)
            p = pb[r, pl.ds(c, L)]
            t = tb[r, pl.ds(c, L)]
            comb = (p * 16.0 + t).astype(jnp.int32) + lane_offs
            plsc.addupdate_scatter(hacc, [comb], ones)

    for c in range(NBINS // L):
        def mbody(l, acc, c=c):
            return acc + hacc[pl.ds(pl.multiple_of(l * NBINS + c * L, L), L)]
        rowbuf[pl.ds(c * L, L)] = lax.fori_loop(0, L, mbody, zero16)

    pltpu.sync_copy(rowbuf, hist_out.at[wid])


_sc_hist = functools.partial(
    pl.kernel,
    mesh=plsc.VectorSubcoreMesh(core_axis_name="c", subcore_axis_name="s"),
    out_type=jax.ShapeDtypeStruct((NW, NBINS), jnp.float32),
    scratch_types=[pltpu.VMEM((CR, W), jnp.float32),
                   pltpu.VMEM((CR, W), jnp.float32),
                   pltpu.VMEM((CR, W), jnp.float32),
                   pltpu.VMEM((CR, W), jnp.float32),
                   pltpu.VMEM((L * NBINS,), jnp.float32),
                   pltpu.VMEM((NBINS,), jnp.float32),
                   pltpu.SemaphoreType.DMA,
                   pltpu.SemaphoreType.DMA,
                   pltpu.SemaphoreType.DMA,
                   pltpu.SemaphoreType.DMA],
    compiler_params=pltpu.CompilerParams(needs_layout_passes=False),
)(_sc_hist_kernel)


def _tc_hist_kernel(pred_ref, true_ref, out_ref, acc_ref):
    step = pl.program_id(0)

    @pl.when(step == 0)
    def _init():
        acc_ref[...] = jnp.zeros_like(acc_ref)

    p = pred_ref[...]
    t = true_ref[...]

    ids = (jax.lax.broadcasted_iota(jnp.int32, (NUM * NUM, W), 0)
           & (NUM - 1)).astype(jnp.float32)
    r = jnp.zeros((NUM * NUM, NUM * NUM), jnp.float32)
    for u in range(TSUB):
        ps = p[u * NUM:(u + 1) * NUM, :]
        ts = t[u * NUM:(u + 1) * NUM, :]
        pr = jnp.broadcast_to(ps[:, None, :],
                              (NUM, NUM, W)).reshape(NUM * NUM, W)
        tr = jnp.broadcast_to(ts[:, None, :],
                              (NUM, NUM, W)).reshape(NUM * NUM, W)
        ap = (pr == ids).astype(jnp.bfloat16)
        at = (tr == ids).astype(jnp.bfloat16)
        r = r + jax.lax.dot_general(ap, at, (((1,), (1,)), ((), ())),
                                    preferred_element_type=jnp.float32)
    acc_ref[...] += r

    @pl.when(step == TGRID - 1)
    def _fin():
        rm = acc_ref[...]
        s0 = jax.lax.broadcasted_iota(jnp.int32, (NBINS, NBINS), 0)
        s1 = jax.lax.broadcasted_iota(jnp.int32, (NBINS, NBINS), 1)
        rm = jnp.where((s0 >> 4) == (s1 >> 4), rm, 0.0)
        e0 = jax.lax.broadcasted_iota(jnp.int32, (NBINS, NUM), 0)
        e1 = jax.lax.broadcasted_iota(jnp.int32, (NBINS, NUM), 1)
        e = ((e0 & (NUM - 1)) == e1).astype(jnp.float32)
        re = jax.lax.dot_general(rm, e, (((1,), (0,)), ((), ())),
                                 preferred_element_type=jnp.float32)
        out_ref[...] = jax.lax.dot_general(e, re, (((0,), (0,)), ((), ())),
                                           preferred_element_type=jnp.float32)


def _tc_epilogue_kernel(hist_ref, jtc_ref, out_ref):
    h = hist_ref[...]
    flat = jnp.sum(h, axis=0, keepdims=True)

    bi = jax.lax.broadcasted_iota(jnp.int32, (NUM, NBINS), 0)
    bm = jax.lax.broadcasted_iota(jnp.int32, (NUM, NBINS), 1)
    d = jnp.where((bm >> 4) == bi, flat, 0.0)
    cm = jax.lax.broadcasted_iota(jnp.int32, (NBINS, NUM), 0)
    cj = jax.lax.broadcasted_iota(jnp.int32, (NBINS, NUM), 1)
    c = ((cm & (NUM - 1)) == cj).astype(jnp.float32)
    j = jax.lax.dot_general(d, c, (((1,), (0,)), ((), ())),
                            preferred_element_type=jnp.float32)
    j = j + jtc_ref[...]

    ri = jax.lax.broadcasted_iota(jnp.int32, (NUM, NUM), 0)
    ci = jax.lax.broadcasted_iota(jnp.int32, (NUM, NUM), 1)
    df = (ri - ci).astype(jnp.float32)
    mse_sum = jnp.sum(j * df * df)
    valid = (ri >= 1) & (ci >= 1)
    inter = jnp.where(valid, j, 0.0)
    pc = jnp.sum(j, axis=1, keepdims=True)
    tc = jnp.sum(j, axis=0, keepdims=True)
    union = pc + tc - inter
    iou = jnp.where(valid & (union != 0.0),
                    inter / jnp.maximum(union, 1e-12), 0.0)
    max_p = jnp.max(iou, axis=1, keepdims=True)
    max_t = jnp.max(iou, axis=0, keepdims=True)
    rv = (jax.lax.broadcasted_iota(jnp.int32, (NUM, 1), 0) >= 1) & (pc > 0)
    cv = (jax.lax.broadcasted_iota(jnp.int32, (1, NUM), 1) >= 1) & (tc > 0)
    loss_p = jnp.sum(jnp.where(rv, 1.0 - max_p, 0.0))
    loss_t = jnp.sum(jnp.where(cv, 1.0 - max_t, 0.0))
    ninst = (jnp.sum(rv.astype(jnp.float32))
             + jnp.sum(cv.astype(jnp.float32)))
    total = mse_sum / (H * W) / 1000.0 + loss_p + loss_t
    out_ref[...] = jnp.reshape(jnp.where(ninst == 0.0, 0.0, total), (1, 1))


def kernel(pred_mask, true_mask):
    hist = _sc_hist(pred_mask, true_mask)
    off = HSC // TBR
    jtc = pl.pallas_call(
        _tc_hist_kernel,
        grid=(TGRID,),
        in_specs=[pl.BlockSpec((TBR, W), lambda i: (i + off, 0)),
                  pl.BlockSpec((TBR, W), lambda i: (i + off, 0))],
        out_specs=pl.BlockSpec((NUM, NUM), lambda i: (0, 0)),
        out_shape=jax.ShapeDtypeStruct((NUM, NUM), jnp.float32),
        scratch_shapes=[pltpu.VMEM((NBINS, NBINS), jnp.float32)],
    )(pred_mask, true_mask)
    out = pl.pallas_call(
        _tc_epilogue_kernel,
        out_shape=jax.ShapeDtypeStruct((1, 1), jnp.float32),
    )(hist, jtc)
    return out[0, 0]


# speedup vs baseline: 1.0438x; 1.0115x over previous
import functools

import jax
import jax.numpy as jnp
from jax import lax
from jax.experimental import pallas as pl
from jax.experimental.pallas import tpu as pltpu
from jax.experimental.pallas import tpu_sc as plsc

NUM = 16
H = 1024
W = 1024
HSC = 512
NBINS = NUM * NUM

_info = plsc.get_sparse_core_info()
NC, NS, L = _info.num_cores, _info.num_subcores, _info.num_lanes
NW = NC * NS
RPW = HSC // NW
VPR = W // 16
VPR_LOG2 = 6
NCH = 2
CR = RPW // NCH
UNROLL = 8

TBR = 128
TGRID = (H - HSC) // TBR
TSUB = TBR // NUM


def _sc_hist_kernel(pred_hbm, true_hbm, hist_out,
                    pbuf0, tbuf0, pbuf1, tbuf1, hacc, rowbuf,
                    sp0, st0, sp1, st1):
    wid = lax.axis_index("s") * NC + lax.axis_index("c")
    pbufs, tbufs = (pbuf0, pbuf1), (tbuf0, tbuf1)
    sems = ((sp0, st0), (sp1, st1))

    def start(k, slot):
        row = wid * RPW + k * CR
        hp = pltpu.async_copy(pred_hbm.at[pl.ds(row, CR)], pbufs[slot],
                              sems[slot][0])
        ht = pltpu.async_copy(true_hbm.at[pl.ds(row, CR)], tbufs[slot],
                              sems[slot][1])
        return hp, ht

    inflight = [None, None]
    inflight[0] = start(0, 0)

    zero16 = jnp.zeros((L,), jnp.float32)

    def zbody(i, _):
        hacc[pl.ds(pl.multiple_of(i * L, L), L)] = zero16
        return 0
    lax.fori_loop(0, (L * NBINS) // L, zbody, 0)

    lane_offs = lax.iota(jnp.int32, L) * NBINS
    ones = jnp.ones((L,), jnp.float32)

    for k in range(NCH):
        slot = k % 2
        if k + 1 < NCH:
            inflight[1 - slot] = start(k + 1, 1 - slot)
        hp, ht = inflight[slot]
        hp.wait()
        ht.wait()
        pb, tb = pbufs[slot], tbufs[slot]

        @plsc.parallel_loop(0, (CR * W) // L, 1, unroll=UNROLL)
        def body(i, pb=pb, tb=tb):
            r = i >> VPR_LOG2
            c = pl.multiple_of((i & (VPR - 1)) * L, L)
            p = pb[r, pl.ds(c, L)]
            t = tb[r, pl.ds(c, L)]
            comb = (p * 16.0 + t).astype(jnp.int32) + lane_offs
            plsc.addupdate_scatter(hacc, [comb], ones)

    for c in range(NBINS // L):
        def mbody(l, acc, c=c):
            return acc + hacc[pl.ds(pl.multiple_of(l * NBINS + c * L, L), L)]
        rowbuf[pl.ds(c * L, L)] = lax.fori_loop(0, L, mbody, zero16)

    pltpu.sync_copy(rowbuf, hist_out.at[wid])


_sc_hist = functools.partial(
    pl.kernel,
    mesh=plsc.VectorSubcoreMesh(core_axis_name="c", subcore_axis_name="s"),
    out_type=jax.ShapeDtypeStruct((NW, NBINS), jnp.float32),
    scratch_types=[pltpu.VMEM((CR, W), jnp.float32),
                   pltpu.VMEM((CR, W), jnp.float32),
                   pltpu.VMEM((CR, W), jnp.float32),
                   pltpu.VMEM((CR, W), jnp.float32),
                   pltpu.VMEM((L * NBINS,), jnp.float32),
                   pltpu.VMEM((NBINS,), jnp.float32),
                   pltpu.SemaphoreType.DMA,
                   pltpu.SemaphoreType.DMA,
                   pltpu.SemaphoreType.DMA,
                   pltpu.SemaphoreType.DMA],
    compiler_params=pltpu.CompilerParams(needs_layout_passes=False),
)(_sc_hist_kernel)


def _tc_hist_kernel(pred_ref, true_ref, out_ref, acc_ref):
    step = pl.program_id(0)

    @pl.when(step == 0)
    def _init():
        acc_ref[...] = jnp.zeros_like(acc_ref)

    p = pred_ref[...]
    t = true_ref[...]

    ids = (jax.lax.broadcasted_iota(jnp.int32, (NUM * NUM, W), 0)
           & (NUM - 1)).astype(jnp.float32)
    r = jnp.zeros((NUM * NUM, NUM * NUM), jnp.float32)
    for u in range(TSUB):
        ps = p[u * NUM:(u + 1) * NUM, :]
        ts = t[u * NUM:(u + 1) * NUM, :]
        pr = jnp.broadcast_to(ps[:, None, :],
                              (NUM, NUM, W)).reshape(NUM * NUM, W)
        tr = jnp.broadcast_to(ts[:, None, :],
                              (NUM, NUM, W)).reshape(NUM * NUM, W)
        ap = (pr == ids).astype(jnp.bfloat16)
        at = (tr == ids).astype(jnp.bfloat16)
        r = r + jax.lax.dot_general(ap, at, (((1,), (1,)), ((), ())),
                                    preferred_element_type=jnp.float32)
    acc_ref[...] += r

    @pl.when(step == TGRID - 1)
    def _fin():
        rm = acc_ref[...]
        s0 = jax.lax.broadcasted_iota(jnp.int32, (NBINS, NBINS), 0)
        s1 = jax.lax.broadcasted_iota(jnp.int32, (NBINS, NBINS), 1)
        rm = jnp.where((s0 >> 4) == (s1 >> 4), rm, 0.0)
        e0 = jax.lax.broadcasted_iota(jnp.int32, (NBINS, NUM), 0)
        e1 = jax.lax.broadcasted_iota(jnp.int32, (NBINS, NUM), 1)
        e = ((e0 & (NUM - 1)) == e1).astype(jnp.float32)
        re = jax.lax.dot_general(rm, e, (((1,), (0,)), ((), ())),
                                 preferred_element_type=jnp.float32)
        out_ref[...] = jax.lax.dot_general(e, re, (((0,), (0,)), ((), ())),
                                           preferred_element_type=jnp.float32)


def _tc_epilogue_kernel(hist_ref, jtc_ref, out_ref):
    h = hist_ref[...]
    flat = jnp.sum(h, axis=0, keepdims=True)

    bi = jax.lax.broadcasted_iota(jnp.int32, (NUM, NBINS), 0)
    bm = jax.lax.broadcasted_iota(jnp.int32, (NUM, NBINS), 1)
    d = jnp.where((bm >> 4) == bi, flat, 0.0)
    cm = jax.lax.broadcasted_iota(jnp.int32, (NBINS, NUM), 0)
    cj = jax.lax.broadcasted_iota(jnp.int32, (NBINS, NUM), 1)
    c = ((cm & (NUM - 1)) == cj).astype(jnp.float32)
    j = jax.lax.dot_general(d, c, (((1,), (0,)), ((), ())),
                            preferred_element_type=jnp.float32)
    j = j + jtc_ref[...]

    ri = jax.lax.broadcasted_iota(jnp.int32, (NUM, NUM), 0)
    ci = jax.lax.broadcasted_iota(jnp.int32, (NUM, NUM), 1)
    df = (ri - ci).astype(jnp.float32)
    mse_sum = jnp.sum(j * df * df)
    valid = (ri >= 1) & (ci >= 1)
    inter = jnp.where(valid, j, 0.0)
    pc = jnp.sum(j, axis=1, keepdims=True)
    tc = jnp.sum(j, axis=0, keepdims=True)
    union = pc + tc - inter
    iou = jnp.where(valid & (union != 0.0),
                    inter / jnp.maximum(union, 1e-12), 0.0)
    max_p = jnp.max(iou, axis=1, keepdims=True)
    max_t = jnp.max(iou, axis=0, keepdims=True)
    rv = (jax.lax.broadcasted_iota(jnp.int32, (NUM, 1), 0) >= 1) & (pc > 0)
    cv = (jax.lax.broadcasted_iota(jnp.int32, (1, NUM), 1) >= 1) & (tc > 0)
    loss_p = jnp.sum(jnp.where(rv, 1.0 - max_p, 0.0))
    loss_t = jnp.sum(jnp.where(cv, 1.0 - max_t, 0.0))
    ninst = (jnp.sum(rv.astype(jnp.float32))
             + jnp.sum(cv.astype(jnp.float32)))
    total = mse_sum / (H * W) / 1000.0 + loss_p + loss_t
    out_ref[...] = jnp.reshape(jnp.where(ninst == 0.0, 0.0, total), (1, 1))


def kernel(pred_mask, true_mask):
    hist = _sc_hist(pred_mask, true_mask)
    off = HSC // TBR
    jtc = pl.pallas_call(
        _tc_hist_kernel,
        grid=(TGRID,),
        in_specs=[pl.BlockSpec((TBR, W), lambda i: (i + off, 0)),
                  pl.BlockSpec((TBR, W), lambda i: (i + off, 0))],
        out_specs=pl.BlockSpec((NUM, NUM), lambda i: (0, 0)),
        out_shape=jax.ShapeDtypeStruct((NUM, NUM), jnp.float32),
        scratch_shapes=[pltpu.VMEM((NBINS, NBINS), jnp.float32)],
    )(pred_mask, true_mask)
    out = pl.pallas_call(
        _tc_epilogue_kernel,
        out_shape=jax.ShapeDtypeStruct((1, 1), jnp.float32),
    )(hist, jtc)
    return out[0, 0]
